# Initial kernel scaffold; baseline (speedup 1.0000x reference)
import jax
import jax.numpy as jnp
from jax.experimental import pallas as pl


def kernel(x, edge_index, W1, b1, W2, b2):
    raise NotImplementedError("write your pallas kernel here")


# speedup vs baseline: 5.6673x; 5.6673x over previous
import functools

import jax
import jax.numpy as jnp
from jax import lax
from jax.experimental import pallas as pl
from jax.experimental.pallas import tpu as pltpu
from jax.experimental.pallas import tpu_sc as plsc

F32 = jnp.float32
I32 = jnp.int32

N = 10000
E = 160000
NP = 10240
EP = 163840
NROWS16 = NP // 16
TILES = 32
EROWS = EP // 128
EROWS_T = EROWS // TILES
NROWS_T = NROWS16 // 16

_mesh = plsc.VectorSubcoreMesh(core_axis_name="c", subcore_axis_name="s")


def _deg_call(dst2d):
    width = 128
    rows_per_tile = NP // 16

    @functools.partial(
        pl.kernel,
        out_type=jax.ShapeDtypeStruct((2, NP, width), F32),
        mesh=_mesh,
        scratch_types=[
            pltpu.VMEM((EROWS_T, 128), I32),
            pltpu.VMEM((128, width), F32),
            pltpu.VMEM((EROWS_T, width), F32),
            pltpu.VMEM_SHARED((NP, width), F32),
        ],
    )
    def k(dst_hbm, out_hbm, dstv, onev, zerov, shared):
        c = lax.axis_index("c")
        s = lax.axis_index("s")
        w = c * 16 + s

        pltpu.sync_copy(dst_hbm.at[pl.ds(w * EROWS_T, EROWS_T)], dstv)

        zeros16 = jnp.zeros((16,), F32)
        ones16 = jnp.ones((16,), F32)

        @pl.loop(0, 128)
        def _(r):
            @pl.loop(0, width // 16)
            def _(j):
                onev[r, pl.ds(j * 16, 16)] = ones16

        @pl.loop(0, EROWS_T)
        def _(r):
            @pl.loop(0, width // 16)
            def _(j):
                zerov[r, pl.ds(j * 16, 16)] = zeros16

        @pl.loop(0, rows_per_tile // EROWS_T)
        def _(t):
            pltpu.sync_copy(
                zerov,
                shared.at[pl.ds(s * rows_per_tile + t * EROWS_T, EROWS_T)],
            )
        plsc.subcore_barrier()

        @pl.loop(0, EROWS_T)
        def _(r):
            pltpu.sync_copy(onev, shared.at[dstv.at[r]], add=True)

        plsc.subcore_barrier()
        pltpu.sync_copy(
            shared.at[pl.ds(s * rows_per_tile, rows_per_tile)],
            out_hbm.at[c, pl.ds(s * rows_per_tile, rows_per_tile)],
        )

    return k(dst2d)


def _accum_call(h, src2d, dst2d):
    width = 128
    rows_per_tile = NP // 16

    @functools.partial(
        pl.kernel,
        out_type=jax.ShapeDtypeStruct((2, NP, width), F32),
        mesh=_mesh,
        scratch_types=[
            pltpu.VMEM((EROWS_T, 128), I32),
            pltpu.VMEM((EROWS_T, 128), I32),
            pltpu.VMEM((128, width), F32),
            pltpu.VMEM_SHARED((NP, width), F32),
            pltpu.SemaphoreType.DMA,
        ],
    )
    def k(h_hbm, src_hbm, dst_hbm, out_hbm, srcv, dstv, rows, shared, sem):
        c = lax.axis_index("c")
        s = lax.axis_index("s")
        w = c * 16 + s

        pltpu.sync_copy(src_hbm.at[pl.ds(w * EROWS_T, EROWS_T)], srcv)
        pltpu.sync_copy(dst_hbm.at[pl.ds(w * EROWS_T, EROWS_T)], dstv)
        pltpu.sync_copy(
            h_hbm.at[pl.ds(s * rows_per_tile, rows_per_tile)],
            shared.at[pl.ds(s * rows_per_tile, rows_per_tile)],
        )
        plsc.subcore_barrier()

        @pl.loop(0, EROWS_T)
        def _(r):
            pltpu.async_copy(h_hbm.at[srcv.at[r]], rows, sem).wait()
            pltpu.sync_copy(rows, shared.at[dstv.at[r]], add=True)

        plsc.subcore_barrier()
        pltpu.sync_copy(
            shared.at[pl.ds(s * rows_per_tile, rows_per_tile)],
            out_hbm.at[c, pl.ds(s * rows_per_tile, rows_per_tile)],
        )

    return k(h, src2d, dst2d)


def _mm1_kernel(x_ref, w_ref, dp_ref, o_ref):
    deg = dp_ref[0, :, 0] + dp_ref[1, :, 0] + 1.0
    dinv = lax.rsqrt(deg)
    u = jnp.dot(x_ref[...], w_ref[...],
                preferred_element_type=F32,
                precision=lax.Precision.HIGHEST)
    o_ref[...] = u * dinv[:, None]


def _mm1_call(xp, W1p, degp):
    bm = 1024
    return pl.pallas_call(
        _mm1_kernel,
        grid=(NP // bm,),
        in_specs=[
            pl.BlockSpec((bm, 128), lambda i: (i, 0)),
            pl.BlockSpec((128, 128), lambda i: (0, 0)),
            pl.BlockSpec((2, bm, 16), lambda i: (0, i, 0)),
        ],
        out_specs=pl.BlockSpec((bm, 128), lambda i: (i, 0)),
        out_shape=jax.ShapeDtypeStruct((NP, 128), F32),
    )(xp, W1p, degp)


def _k2_kernel(ap_ref, h_ref, dp_ref, w_ref, b_ref, o_ref):
    deg = dp_ref[0, :, 0] + dp_ref[1, :, 0] + 1.0
    dinv = lax.rsqrt(deg)
    a = ap_ref[0] + ap_ref[1] - h_ref[...]
    h1 = jnp.maximum(a * dinv[:, None] + b_ref[...], 0.0)
    g = jnp.dot(h1, w_ref[...],
                preferred_element_type=F32,
                precision=lax.Precision.HIGHEST)
    o_ref[...] = g * dinv[:, None]


def _k2_call(a1p, hprime, degp, W2p, b1p):
    bm = 1024
    return pl.pallas_call(
        _k2_kernel,
        grid=(NP // bm,),
        in_specs=[
            pl.BlockSpec((2, bm, 128), lambda i: (0, i, 0)),
            pl.BlockSpec((bm, 128), lambda i: (i, 0)),
            pl.BlockSpec((2, bm, 16), lambda i: (0, i, 0)),
            pl.BlockSpec((128, 128), lambda i: (0, 0)),
            pl.BlockSpec((1, 128), lambda i: (0, 0)),
        ],
        out_specs=pl.BlockSpec((bm, 128), lambda i: (i, 0)),
        out_shape=jax.ShapeDtypeStruct((NP, 128), F32),
    )(a1p, hprime, degp, W2p, b1p)


def _zk_kernel(ap_ref, g_ref, dp_ref, b_ref, o_ref):
    deg = dp_ref[0, :, 0] + dp_ref[1, :, 0] + 1.0
    dinv = lax.rsqrt(deg)
    a = ap_ref[0, :, :32] + ap_ref[1, :, :32] - g_ref[:, :32]
    o_ref[...] = a * dinv[:, None] + b_ref[...]


def _zk_call(a2p, gprime, degp, b2):
    bm = 1024
    return pl.pallas_call(
        _zk_kernel,
        grid=(NP // bm,),
        in_specs=[
            pl.BlockSpec((2, bm, 128), lambda i: (0, i, 0)),
            pl.BlockSpec((bm, 128), lambda i: (i, 0)),
            pl.BlockSpec((2, bm, 16), lambda i: (0, i, 0)),
            pl.BlockSpec((1, 32), lambda i: (0, 0)),
        ],
        out_specs=pl.BlockSpec((bm, 32), lambda i: (i, 0)),
        out_shape=jax.ShapeDtypeStruct((NP, 32), F32),
    )(a2p, gprime, degp, b2)


def _decode_kernel(zr_ref, zc_ref, o_ref):
    t = lax.dot_general(
        zr_ref[...], zc_ref[...],
        (((1,), (1,)), ((), ())),
        preferred_element_type=F32,
        precision=lax.Precision.HIGHEST,
    )
    o_ref[...] = 0.5 * jnp.tanh(0.5 * t) + 0.5


def _decode_call(z):
    bm, bn = 256, 5120
    return pl.pallas_call(
        _decode_kernel,
        grid=(pl.cdiv(N, bm), pl.cdiv(N, bn)),
        in_specs=[
            pl.BlockSpec((bm, 32), lambda i, j: (i, 0)),
            pl.BlockSpec((bn, 32), lambda i, j: (j, 0)),
        ],
        out_specs=pl.BlockSpec((bm, bn), lambda i, j: (i, j)),
        out_shape=jax.ShapeDtypeStruct((N, N), F32),
    )(z, z)


def kernel(x, edge_index, W1, b1, W2, b2):
    ei = edge_index.astype(I32)
    pad = jnp.full((EP - E,), N, I32)
    src2d = jnp.concatenate([ei[0], pad]).reshape(EROWS, 128)
    dst2d = jnp.concatenate([ei[1], pad]).reshape(EROWS, 128)
    xp = jnp.concatenate([x, jnp.zeros((NP - N, x.shape[1]), F32)], axis=0)
    W1p = jnp.concatenate([W1, jnp.zeros((128, 64), F32)], axis=1)
    W2p = jnp.zeros((128, 128), F32).at[:64, :32].set(W2)
    b1p = jnp.concatenate([b1, jnp.zeros((64,), F32)]).reshape(1, 128)

    degp = _deg_call(dst2d)[:, :, :16]
    hprime = _mm1_call(xp, W1p, degp)
    a1p = _accum_call(hprime, src2d, dst2d)
    gprime = _k2_call(a1p, hprime, degp, W2p, b1p)
    a2p = _accum_call(gprime, src2d, dst2d)
    z = _zk_call(a2p, gprime, degp, b2.reshape(1, 32))
    return _decode_call(z)


# speedup vs baseline: 5.7382x; 1.0125x over previous
import functools

import jax
import jax.numpy as jnp
from jax import lax
from jax.experimental import pallas as pl
from jax.experimental.pallas import tpu as pltpu
from jax.experimental.pallas import tpu_sc as plsc

F32 = jnp.float32
I32 = jnp.int32

N = 10000
E = 160000
NP = 10240
EP = 163840
NROWS16 = NP // 16
TILES = 32
EROWS = EP // 128
EROWS_T = EROWS // TILES
NROWS_T = NROWS16 // 16

_mesh = plsc.VectorSubcoreMesh(core_axis_name="c", subcore_axis_name="s")


def _deg_call(dst2d):
    width = 128
    rows_per_tile = NP // 16

    @functools.partial(
        pl.kernel,
        out_type=jax.ShapeDtypeStruct((2, NP, width), F32),
        mesh=_mesh,
        scratch_types=[
            pltpu.VMEM((EROWS_T, 128), I32),
            pltpu.VMEM((128, width), F32),
            pltpu.VMEM((EROWS_T, width), F32),
            pltpu.VMEM_SHARED((NP, width), F32),
        ],
    )
    def k(dst_hbm, out_hbm, dstv, onev, zerov, shared):
        c = lax.axis_index("c")
        s = lax.axis_index("s")
        w = c * 16 + s

        pltpu.sync_copy(dst_hbm.at[pl.ds(w * EROWS_T, EROWS_T)], dstv)

        zeros16 = jnp.zeros((16,), F32)
        ones16 = jnp.ones((16,), F32)

        @pl.loop(0, 128)
        def _(r):
            @pl.loop(0, width // 16)
            def _(j):
                onev[r, pl.ds(j * 16, 16)] = ones16

        @pl.loop(0, EROWS_T)
        def _(r):
            @pl.loop(0, width // 16)
            def _(j):
                zerov[r, pl.ds(j * 16, 16)] = zeros16

        @pl.loop(0, rows_per_tile // EROWS_T)
        def _(t):
            pltpu.sync_copy(
                zerov,
                shared.at[pl.ds(s * rows_per_tile + t * EROWS_T, EROWS_T)],
            )
        plsc.subcore_barrier()

        @pl.loop(0, EROWS_T)
        def _(r):
            pltpu.sync_copy(onev, shared.at[dstv.at[r]], add=True)

        plsc.subcore_barrier()
        pltpu.sync_copy(
            shared.at[pl.ds(s * rows_per_tile, rows_per_tile)],
            out_hbm.at[c, pl.ds(s * rows_per_tile, rows_per_tile)],
        )

    return k(dst2d)


def _accum_call(h, src2d, dst2d):
    width = 128
    rows_per_tile = NP // 16

    @functools.partial(
        pl.kernel,
        out_type=jax.ShapeDtypeStruct((2, NP, width), F32),
        mesh=_mesh,
        scratch_types=[
            pltpu.VMEM((EROWS_T, 128), I32),
            pltpu.VMEM((EROWS_T, 128), I32),
            pltpu.VMEM((128, width), F32),
            pltpu.VMEM((128, width), F32),
            pltpu.VMEM_SHARED((NP, width), F32),
            pltpu.SemaphoreType.DMA,
            pltpu.SemaphoreType.DMA,
        ],
    )
    def k(h_hbm, src_hbm, dst_hbm, out_hbm, srcv, dstv,
          rows0, rows1, shared, gs0, gs1):
        c = lax.axis_index("c")
        s = lax.axis_index("s")
        w = c * 16 + s
        rows = [rows0, rows1]
        gsem = [gs0, gs1]

        pltpu.sync_copy(src_hbm.at[pl.ds(w * EROWS_T, EROWS_T)], srcv)
        pltpu.sync_copy(dst_hbm.at[pl.ds(w * EROWS_T, EROWS_T)], dstv)
        pltpu.sync_copy(
            h_hbm.at[pl.ds(s * rows_per_tile, rows_per_tile)],
            shared.at[pl.ds(s * rows_per_tile, rows_per_tile)],
        )
        plsc.subcore_barrier()

        @pl.loop(0, EROWS_T, step=2)
        def _(r):
            gh = [
                pltpu.async_copy(h_hbm.at[srcv.at[r + b]], rows[b], gsem[b])
                for b in range(2)
            ]
            for b in range(2):
                gh[b].wait()
                pltpu.sync_copy(rows[b], shared.at[dstv.at[r + b]], add=True)

        plsc.subcore_barrier()
        pltpu.sync_copy(
            shared.at[pl.ds(s * rows_per_tile, rows_per_tile)],
            out_hbm.at[c, pl.ds(s * rows_per_tile, rows_per_tile)],
        )

    return k(h, src2d, dst2d)


def _mm1_kernel(x_ref, w_ref, dp_ref, o_ref):
    deg = dp_ref[0, :, 0] + dp_ref[1, :, 0] + 1.0
    dinv = lax.rsqrt(deg)
    u = jnp.dot(x_ref[...], w_ref[...],
                preferred_element_type=F32,
                precision=lax.Precision.HIGHEST)
    o_ref[...] = u * dinv[:, None]


def _mm1_call(xp, W1p, degp):
    bm = 1024
    return pl.pallas_call(
        _mm1_kernel,
        grid=(NP // bm,),
        in_specs=[
            pl.BlockSpec((bm, 128), lambda i: (i, 0)),
            pl.BlockSpec((128, 128), lambda i: (0, 0)),
            pl.BlockSpec((2, bm, 16), lambda i: (0, i, 0)),
        ],
        out_specs=pl.BlockSpec((bm, 128), lambda i: (i, 0)),
        out_shape=jax.ShapeDtypeStruct((NP, 128), F32),
    )(xp, W1p, degp)


def _k2_kernel(ap_ref, h_ref, dp_ref, w_ref, b_ref, o_ref):
    deg = dp_ref[0, :, 0] + dp_ref[1, :, 0] + 1.0
    dinv = lax.rsqrt(deg)
    a = ap_ref[0] + ap_ref[1] - h_ref[...]
    h1 = jnp.maximum(a * dinv[:, None] + b_ref[...], 0.0)
    g = jnp.dot(h1, w_ref[...],
                preferred_element_type=F32,
                precision=lax.Precision.HIGHEST)
    o_ref[...] = g * dinv[:, None]


def _k2_call(a1p, hprime, degp, W2p, b1p):
    bm = 1024
    return pl.pallas_call(
        _k2_kernel,
        grid=(NP // bm,),
        in_specs=[
            pl.BlockSpec((2, bm, 128), lambda i: (0, i, 0)),
            pl.BlockSpec((bm, 128), lambda i: (i, 0)),
            pl.BlockSpec((2, bm, 16), lambda i: (0, i, 0)),
            pl.BlockSpec((128, 128), lambda i: (0, 0)),
            pl.BlockSpec((1, 128), lambda i: (0, 0)),
        ],
        out_specs=pl.BlockSpec((bm, 128), lambda i: (i, 0)),
        out_shape=jax.ShapeDtypeStruct((NP, 128), F32),
    )(a1p, hprime, degp, W2p, b1p)


def _zk_kernel(ap_ref, g_ref, dp_ref, b_ref, o_ref):
    deg = dp_ref[0, :, 0] + dp_ref[1, :, 0] + 1.0
    dinv = lax.rsqrt(deg)
    a = ap_ref[0, :, :32] + ap_ref[1, :, :32] - g_ref[:, :32]
    o_ref[...] = a * dinv[:, None] + b_ref[...]


def _zk_call(a2p, gprime, degp, b2):
    bm = 1024
    return pl.pallas_call(
        _zk_kernel,
        grid=(NP // bm,),
        in_specs=[
            pl.BlockSpec((2, bm, 128), lambda i: (0, i, 0)),
            pl.BlockSpec((bm, 128), lambda i: (i, 0)),
            pl.BlockSpec((2, bm, 16), lambda i: (0, i, 0)),
            pl.BlockSpec((1, 32), lambda i: (0, 0)),
        ],
        out_specs=pl.BlockSpec((bm, 32), lambda i: (i, 0)),
        out_shape=jax.ShapeDtypeStruct((NP, 32), F32),
    )(a2p, gprime, degp, b2)


def _decode_kernel(zr_ref, zc_ref, o_ref):
    t = lax.dot_general(
        zr_ref[...], zc_ref[...],
        (((1,), (1,)), ((), ())),
        preferred_element_type=F32,
        precision=lax.Precision.HIGHEST,
    )
    o_ref[...] = 0.5 * jnp.tanh(0.5 * t) + 0.5


def _decode_call(z):
    bm, bn = 256, 5120
    return pl.pallas_call(
        _decode_kernel,
        grid=(pl.cdiv(N, bm), pl.cdiv(N, bn)),
        in_specs=[
            pl.BlockSpec((bm, 32), lambda i, j: (i, 0)),
            pl.BlockSpec((bn, 32), lambda i, j: (j, 0)),
        ],
        out_specs=pl.BlockSpec((bm, bn), lambda i, j: (i, j)),
        out_shape=jax.ShapeDtypeStruct((N, N), F32),
    )(z, z)


def kernel(x, edge_index, W1, b1, W2, b2):
    ei = edge_index.astype(I32)
    pad = jnp.full((EP - E,), N, I32)
    src2d = jnp.concatenate([ei[0], pad]).reshape(EROWS, 128)
    dst2d = jnp.concatenate([ei[1], pad]).reshape(EROWS, 128)
    xp = jnp.concatenate([x, jnp.zeros((NP - N, x.shape[1]), F32)], axis=0)
    W1p = jnp.concatenate([W1, jnp.zeros((128, 64), F32)], axis=1)
    W2p = jnp.zeros((128, 128), F32).at[:64, :32].set(W2)
    b1p = jnp.concatenate([b1, jnp.zeros((64,), F32)]).reshape(1, 128)

    degp = _deg_call(dst2d)[:, :, :16]
    hprime = _mm1_call(xp, W1p, degp)
    a1p = _accum_call(hprime, src2d, dst2d)
    gprime = _k2_call(a1p, hprime, degp, W2p, b1p)
    a2p = _accum_call(gprime, src2d, dst2d)
    z = _zk_call(a2p, gprime, degp, b2.reshape(1, 32))
    return _decode_call(z)


# speedup vs baseline: 5.7495x; 1.0020x over previous
import functools

import jax
import jax.numpy as jnp
from jax import lax
from jax.experimental import pallas as pl
from jax.experimental.pallas import tpu as pltpu
from jax.experimental.pallas import tpu_sc as plsc

F32 = jnp.float32
I32 = jnp.int32

N = 10000
E = 160000
NP = 10240
EP = 163840
NROWS16 = NP // 16
TILES = 32
EROWS = EP // 128
EROWS_T = EROWS // TILES
NROWS_T = NROWS16 // 16

_mesh = plsc.VectorSubcoreMesh(core_axis_name="c", subcore_axis_name="s")


def _deg_call(dst2d):
    width = 128
    rows_per_tile = NP // 16

    @functools.partial(
        pl.kernel,
        out_type=jax.ShapeDtypeStruct((2, NP, width), F32),
        mesh=_mesh,
        scratch_types=[
            pltpu.VMEM((EROWS_T, 128), I32),
            pltpu.VMEM((128, width), F32),
            pltpu.VMEM((EROWS_T, width), F32),
            pltpu.VMEM_SHARED((NP, width), F32),
        ],
    )
    def k(dst_hbm, out_hbm, dstv, onev, zerov, shared):
        c = lax.axis_index("c")
        s = lax.axis_index("s")
        w = c * 16 + s

        pltpu.sync_copy(dst_hbm.at[pl.ds(w * EROWS_T, EROWS_T)], dstv)

        zeros16 = jnp.zeros((16,), F32)
        ones16 = jnp.ones((16,), F32)

        @pl.loop(0, 128)
        def _(r):
            @pl.loop(0, width // 16)
            def _(j):
                onev[r, pl.ds(j * 16, 16)] = ones16

        @pl.loop(0, EROWS_T)
        def _(r):
            @pl.loop(0, width // 16)
            def _(j):
                zerov[r, pl.ds(j * 16, 16)] = zeros16

        @pl.loop(0, rows_per_tile // EROWS_T)
        def _(t):
            pltpu.sync_copy(
                zerov,
                shared.at[pl.ds(s * rows_per_tile + t * EROWS_T, EROWS_T)],
            )
        plsc.subcore_barrier()

        @pl.loop(0, EROWS_T)
        def _(r):
            pltpu.sync_copy(onev, shared.at[dstv.at[r]], add=True)

        plsc.subcore_barrier()
        pltpu.sync_copy(
            shared.at[pl.ds(s * rows_per_tile, rows_per_tile)],
            out_hbm.at[c, pl.ds(s * rows_per_tile, rows_per_tile)],
        )

    return k(dst2d)


def _accum_call(h, src2d, dst2d):
    width = 128
    rows_per_tile = NP // 16

    @functools.partial(
        pl.kernel,
        out_type=jax.ShapeDtypeStruct((2, NP, width), F32),
        mesh=_mesh,
        scratch_types=[
            pltpu.VMEM((EROWS_T, 128), I32),
            pltpu.VMEM((EROWS_T, 128), I32),
            pltpu.VMEM((128, width), F32),
            pltpu.VMEM((128, width), F32),
            pltpu.VMEM_SHARED((NP, width), F32),
            pltpu.SemaphoreType.DMA,
            pltpu.SemaphoreType.DMA,
            pltpu.SemaphoreType.DMA,
            pltpu.SemaphoreType.DMA,
        ],
    )
    def k(h_hbm, src_hbm, dst_hbm, out_hbm, srcv, dstv,
          rows0, rows1, shared, gs0, gs1, ss0, ss1):
        c = lax.axis_index("c")
        s = lax.axis_index("s")
        w = c * 16 + s
        rows = [rows0, rows1]
        gsem = [gs0, gs1]
        ssem = [ss0, ss1]

        pltpu.sync_copy(src_hbm.at[pl.ds(w * EROWS_T, EROWS_T)], srcv)
        pltpu.sync_copy(dst_hbm.at[pl.ds(w * EROWS_T, EROWS_T)], dstv)
        pltpu.sync_copy(
            h_hbm.at[pl.ds(s * rows_per_tile, rows_per_tile)],
            shared.at[pl.ds(s * rows_per_tile, rows_per_tile)],
        )
        plsc.subcore_barrier()

        @pl.loop(0, EROWS_T, step=2)
        def _(r):
            gh = [
                pltpu.async_copy(h_hbm.at[srcv.at[r + b]], rows[b], gsem[b])
                for b in range(2)
            ]
            sh = []
            for b in range(2):
                gh[b].wait()
                sh.append(
                    pltpu.async_copy(
                        rows[b], shared.at[dstv.at[r + b]], ssem[b], add=True
                    )
                )
            for b in range(2):
                sh[b].wait()

        plsc.subcore_barrier()
        pltpu.sync_copy(
            shared.at[pl.ds(s * rows_per_tile, rows_per_tile)],
            out_hbm.at[c, pl.ds(s * rows_per_tile, rows_per_tile)],
        )

    return k(h, src2d, dst2d)


def _mm1_kernel(x_ref, w_ref, dp_ref, o_ref):
    deg = dp_ref[0, :, 0] + dp_ref[1, :, 0] + 1.0
    dinv = lax.rsqrt(deg)
    u = jnp.dot(x_ref[...], w_ref[...],
                preferred_element_type=F32,
                precision=lax.Precision.HIGHEST)
    o_ref[...] = u * dinv[:, None]


def _mm1_call(xp, W1p, degp):
    bm = 1024
    return pl.pallas_call(
        _mm1_kernel,
        grid=(NP // bm,),
        in_specs=[
            pl.BlockSpec((bm, 128), lambda i: (i, 0)),
            pl.BlockSpec((128, 128), lambda i: (0, 0)),
            pl.BlockSpec((2, bm, 16), lambda i: (0, i, 0)),
        ],
        out_specs=pl.BlockSpec((bm, 128), lambda i: (i, 0)),
        out_shape=jax.ShapeDtypeStruct((NP, 128), F32),
    )(xp, W1p, degp)


def _k2_kernel(ap_ref, h_ref, dp_ref, w_ref, b_ref, o_ref):
    deg = dp_ref[0, :, 0] + dp_ref[1, :, 0] + 1.0
    dinv = lax.rsqrt(deg)
    a = ap_ref[0] + ap_ref[1] - h_ref[...]
    h1 = jnp.maximum(a * dinv[:, None] + b_ref[...], 0.0)
    g = jnp.dot(h1, w_ref[...],
                preferred_element_type=F32,
                precision=lax.Precision.HIGHEST)
    o_ref[...] = g * dinv[:, None]


def _k2_call(a1p, hprime, degp, W2p, b1p):
    bm = 1024
    return pl.pallas_call(
        _k2_kernel,
        grid=(NP // bm,),
        in_specs=[
            pl.BlockSpec((2, bm, 128), lambda i: (0, i, 0)),
            pl.BlockSpec((bm, 128), lambda i: (i, 0)),
            pl.BlockSpec((2, bm, 16), lambda i: (0, i, 0)),
            pl.BlockSpec((128, 128), lambda i: (0, 0)),
            pl.BlockSpec((1, 128), lambda i: (0, 0)),
        ],
        out_specs=pl.BlockSpec((bm, 128), lambda i: (i, 0)),
        out_shape=jax.ShapeDtypeStruct((NP, 128), F32),
    )(a1p, hprime, degp, W2p, b1p)


def _zk_kernel(ap_ref, g_ref, dp_ref, b_ref, o_ref):
    deg = dp_ref[0, :, 0] + dp_ref[1, :, 0] + 1.0
    dinv = lax.rsqrt(deg)
    a = ap_ref[0, :, :32] + ap_ref[1, :, :32] - g_ref[:, :32]
    o_ref[...] = a * dinv[:, None] + b_ref[...]


def _zk_call(a2p, gprime, degp, b2):
    bm = 1024
    return pl.pallas_call(
        _zk_kernel,
        grid=(NP // bm,),
        in_specs=[
            pl.BlockSpec((2, bm, 128), lambda i: (0, i, 0)),
            pl.BlockSpec((bm, 128), lambda i: (i, 0)),
            pl.BlockSpec((2, bm, 16), lambda i: (0, i, 0)),
            pl.BlockSpec((1, 32), lambda i: (0, 0)),
        ],
        out_specs=pl.BlockSpec((bm, 32), lambda i: (i, 0)),
        out_shape=jax.ShapeDtypeStruct((NP, 32), F32),
    )(a2p, gprime, degp, b2)


def _decode_kernel(zr_ref, zc_ref, o_ref):
    t = lax.dot_general(
        zr_ref[...], zc_ref[...],
        (((1,), (1,)), ((), ())),
        preferred_element_type=F32,
        precision=lax.Precision.HIGHEST,
    )
    o_ref[...] = 0.5 * jnp.tanh(0.5 * t) + 0.5


def _decode_call(z):
    bm, bn = 256, 5120
    return pl.pallas_call(
        _decode_kernel,
        grid=(pl.cdiv(N, bm), pl.cdiv(N, bn)),
        in_specs=[
            pl.BlockSpec((bm, 32), lambda i, j: (i, 0)),
            pl.BlockSpec((bn, 32), lambda i, j: (j, 0)),
        ],
        out_specs=pl.BlockSpec((bm, bn), lambda i, j: (i, j)),
        out_shape=jax.ShapeDtypeStruct((N, N), F32),
    )(z, z)


def kernel(x, edge_index, W1, b1, W2, b2):
    ei = edge_index.astype(I32)
    pad = jnp.full((EP - E,), N, I32)
    src2d = jnp.concatenate([ei[0], pad]).reshape(EROWS, 128)
    dst2d = jnp.concatenate([ei[1], pad]).reshape(EROWS, 128)
    xp = jnp.concatenate([x, jnp.zeros((NP - N, x.shape[1]), F32)], axis=0)
    W1p = jnp.concatenate([W1, jnp.zeros((128, 64), F32)], axis=1)
    W2p = jnp.zeros((128, 128), F32).at[:64, :32].set(W2)
    b1p = jnp.concatenate([b1, jnp.zeros((64,), F32)]).reshape(1, 128)

    degp = _deg_call(dst2d)[:, :, :16]
    hprime = _mm1_call(xp, W1p, degp)
    a1p = _accum_call(hprime, src2d, dst2d)
    gprime = _k2_call(a1p, hprime, degp, W2p, b1p)
    a2p = _accum_call(gprime, src2d, dst2d)
    z = _zk_call(a2p, gprime, degp, b2.reshape(1, 32))
    return _decode_call(z)
